# Initial kernel scaffold; baseline (speedup 1.0000x reference)
import functools

import jax
import jax.numpy as jnp
from jax import lax
from jax.experimental import pallas as pl
from jax.experimental.pallas import tpu as pltpu
from jax.experimental.pallas import tpu_sc as plsc

N = 10000
E = 320000
D = 128
G = 64
H = 128
EPS = 1e-5

NC = 2
NS = 16
NW = NC * NS
CHUNK = 128
EPAD = 323584
EROWS = EPAD // CHUNK
TPW = EROWS // NW
NACC = 10016
ZPW = NACC // NS
NSRC = 10048

_HI = lax.Precision.HIGHEST


def _mm(a, b):
    return lax.dot_general(a, b, (((1,), (0,)), ((), ())),
                           precision=_HI, preferred_element_type=jnp.float32)


def _mmT(a, b):
    return lax.dot_general(a, b, (((0,), (0,)), ((), ())),
                           precision=_HI, preferred_element_type=jnp.float32)


def _leaky(v):
    return jnp.where(v >= 0, v, 0.01 * v)


def _onehot(batch2):
    st = (lax.broadcasted_iota(jnp.int32, (N, G), 1) == batch2)
    st = st.astype(jnp.float32)
    cnt = jnp.maximum(_mmT(st, jnp.ones((N, 1), jnp.float32)), 1.0)
    return st, cnt


def _gnorm(v, st, cnt, w, b, ms):
    mean = _mmT(st, v) / cnt
    out = v - ms * _mm(st, mean)
    var = _mmT(st, out * out) / cnt
    rstd = lax.rsqrt(var + EPS)
    return w * out * _mm(st, rstd) + b


_MESH = plsc.VectorSubcoreMesh(core_axis_name="c", subcore_axis_name="s")


@functools.partial(
    pl.kernel,
    out_type=jax.ShapeDtypeStruct((NC, NACC, H), jnp.float32),
    mesh=_MESH,
    scratch_types=[
        pltpu.VMEM((TPW, CHUNK), jnp.int32),
        pltpu.VMEM((TPW, CHUNK), jnp.int32),
        pltpu.VMEM((CHUNK, H), jnp.float32),
        pltpu.VMEM((CHUNK, H), jnp.float32),
        pltpu.VMEM_SHARED((NACC, H), jnp.float32),
        pltpu.SemaphoreType.DMA,
        pltpu.SemaphoreType.DMA,
    ],
)
def _sc_msg(xs_hbm, srcr_hbm, dstr_hbm, zeros_hbm, out_hbm,
            src_v, dst_v, row_a, row_b, acc, sem_a, sem_b):
    c = lax.axis_index("c")
    s = lax.axis_index("s")
    wid = s * NC + c
    pltpu.sync_copy(zeros_hbm.at[pl.ds(s * ZPW, ZPW)], acc.at[pl.ds(s * ZPW, ZPW)])
    pltpu.sync_copy(srcr_hbm.at[pl.ds(wid * TPW, TPW)], src_v)
    pltpu.sync_copy(dstr_hbm.at[pl.ds(wid * TPW, TPW)], dst_v)
    plsc.subcore_barrier()

    def _start(j, buf, sem):
        pltpu.async_copy(xs_hbm.at[src_v.at[j]], buf, sem)

    def _wait(j, buf, sem):
        pltpu.make_async_copy(xs_hbm.at[src_v.at[j]], buf, sem).wait()

    def _scat(j, buf):
        pltpu.sync_copy(buf, acc.at[dst_v.at[j]], add=True)

    _start(0, row_a, sem_a)

    def body(t, carry):
        j = 2 * t
        _start(j + 1, row_b, sem_b)
        _wait(j, row_a, sem_a)
        _scat(j, row_a)
        _start(j + 2, row_a, sem_a)
        _wait(j + 1, row_b, sem_b)
        _scat(j + 1, row_b)
        return carry

    lax.fori_loop(0, (TPW - 1) // 2, body, 0)
    _wait(TPW - 1, row_a, sem_a)
    _scat(TPW - 1, row_a)

    plsc.subcore_barrier()
    pltpu.sync_copy(acc.at[pl.ds(s * ZPW, ZPW)],
                    out_hbm.at[c, pl.ds(s * ZPW, ZPW)])


@functools.partial(
    pl.kernel,
    out_type=jax.ShapeDtypeStruct((NC, NACC, 16), jnp.float32),
    mesh=_MESH,
    scratch_types=[
        pltpu.VMEM((TPW, CHUNK), jnp.int32),
        pltpu.VMEM((CHUNK, 16), jnp.float32),
        pltpu.VMEM_SHARED((NACC, 16), jnp.float32),
    ],
)
def _sc_deg(dstr_hbm, zeros_hbm, ones_hbm, out_hbm, dst_v, ones_v, acc):
    c = lax.axis_index("c")
    s = lax.axis_index("s")
    wid = s * NC + c
    pltpu.sync_copy(zeros_hbm.at[pl.ds(s * ZPW, ZPW)], acc.at[pl.ds(s * ZPW, ZPW)])
    pltpu.sync_copy(dstr_hbm.at[pl.ds(wid * TPW, TPW)], dst_v)
    pltpu.sync_copy(ones_hbm, ones_v)
    plsc.subcore_barrier()

    def body(j, carry):
        pltpu.sync_copy(ones_v, acc.at[dst_v.at[j]], add=True)
        return carry

    lax.fori_loop(0, TPW, body, 0)

    plsc.subcore_barrier()
    pltpu.sync_copy(acc.at[pl.ds(s * ZPW, ZPW)],
                    out_hbm.at[c, pl.ds(s * ZPW, ZPW)])


def _tc1_body(x_ref, pl_ref, batch_ref, deg2_ref, gnw_ref, gnb_ref, gnms_ref,
              w1_ref, xs_ref, dinv_ref):
    x0 = jnp.concatenate(
        [x_ref[...], pl_ref[...], jnp.zeros((N, 127), jnp.float32)], axis=1)
    st, cnt = _onehot(batch_ref[...])
    gn = _gnorm(x0, st, cnt, gnw_ref[...], gnb_ref[...], gnms_ref[...])
    xw = _mm(gn, w1_ref[...])
    deg2 = deg2_ref[...]
    deg = deg2[0, :N, 0:1] + deg2[1, :N, 0:1] + 1.0
    dinv = lax.rsqrt(deg)
    xs = xw * dinv
    xs_ref[...] = jnp.concatenate([xs, jnp.zeros((NSRC - N, H), jnp.float32)], axis=0)
    dinv_ref[...] = dinv


def _tc2_body(p_ref, xs1_ref, dinv_ref, b1_ref, batch_ref, gnw_ref, gnb_ref,
              gnms_ref, w2_ref, xs2_ref, h1_ref):
    pv = p_ref[...]
    xs1 = xs1_ref[...][:N]
    dinv = dinv_ref[...]
    h1 = _leaky(dinv * (pv[0, :N, :] + pv[1, :N, :] + xs1) + b1_ref[...])
    h1_ref[...] = h1
    st, cnt = _onehot(batch_ref[...])
    gn = _gnorm(h1, st, cnt, gnw_ref[...], gnb_ref[...], gnms_ref[...])
    xs2 = _mm(gn, w2_ref[...]) * dinv
    xs2_ref[...] = jnp.concatenate([xs2, jnp.zeros((NSRC - N, H), jnp.float32)], axis=0)


def _bn(v, g, b):
    m = jnp.mean(v, axis=0, keepdims=True)
    var = jnp.mean((v - m) ** 2, axis=0, keepdims=True)
    return g * (v - m) * lax.rsqrt(var + EPS) + b


def _tc3_body(p_ref, xs2_ref, dinv_ref, b2_ref, h1_ref, batch_ref,
              bn1g_ref, bn1b_ref, fw1_ref, fb1_ref, bn2g_ref, bn2b_ref,
              fw2_ref, fb2_ref, y_ref):
    pv = p_ref[...]
    xs2 = xs2_ref[...][:N]
    dinv = dinv_ref[...]
    h2 = _leaky(dinv * (pv[0, :N, :] + pv[1, :N, :] + xs2) + b2_ref[...])
    h12 = jnp.concatenate([h1_ref[...], h2], axis=1)
    st, cnt = _onehot(batch_ref[...])
    pooled = _mmT(st, h12) / cnt
    y1 = _leaky(_mm(_bn(pooled, bn1g_ref[...], bn1b_ref[...]), fw1_ref[...])
                + fb1_ref[...])
    y_ref[...] = _mm(_bn(y1, bn2g_ref[...], bn2b_ref[...]), fw2_ref[...]) + fb2_ref[...]


def _tc1(x, pl2, batch2, deg2, gnw, gnb, gnms, w1p):
    return pl.pallas_call(
        _tc1_body,
        out_shape=[jax.ShapeDtypeStruct((NSRC, H), jnp.float32),
                   jax.ShapeDtypeStruct((N, 1), jnp.float32)],
    )(x, pl2, batch2, deg2, gnw, gnb, gnms, w1p)


def _tc2(p1, xs1, dinv, b1r, batch2, gnw, gnb, gnms, w2):
    return pl.pallas_call(
        _tc2_body,
        out_shape=[jax.ShapeDtypeStruct((NSRC, H), jnp.float32),
                   jax.ShapeDtypeStruct((N, H), jnp.float32)],
    )(p1, xs1, dinv, b1r, batch2, gnw, gnb, gnms, w2)


def _tc3(p2, xs2, dinv, b2r, h1, batch2, bn1g, bn1b, fw1, fb1, bn2g, bn2b,
         fw2, fb2):
    return pl.pallas_call(
        _tc3_body,
        out_shape=jax.ShapeDtypeStruct((G, 1), jnp.float32),
    )(p2, xs2, dinv, b2r, h1, batch2, bn1g, bn1b, fw1, fb1, bn2g, bn2b,
      fw2, fb2)


def _row(v):
    return v.reshape(1, -1).astype(jnp.float32)


def kernel(x, pLDDT, edge_index, batch, gn1_w, gn1_b, gn1_ms, W1, b1,
           gn2_w, gn2_b, gn2_ms, W2, b2, bn1_g, bn1_b, fcW1, fcb1,
           bn2_g, bn2_b, fcW2, fcb2):
    npad = EPAD - E
    pad_src = N + (jnp.arange(npad, dtype=jnp.int32) % (NSRC - N))
    pad_dst = N + (jnp.arange(npad, dtype=jnp.int32) % (NACC - N))
    srcr = jnp.concatenate([edge_index[0], pad_src]).reshape(EROWS, CHUNK)
    dstr = jnp.concatenate([edge_index[1], pad_dst]).reshape(EROWS, CHUNK)
    batch2 = batch.reshape(N, 1)
    pl2 = pLDDT.reshape(N, 1)

    zpad = jnp.zeros((256 - (D + 1),), jnp.float32)
    gn1w = jnp.concatenate([gn1_w, zpad]).reshape(1, 256)
    gn1b = jnp.concatenate([gn1_b, zpad]).reshape(1, 256)
    gn1ms = jnp.concatenate([gn1_ms, zpad]).reshape(1, 256)
    w1p = jnp.concatenate([W1, jnp.zeros((256 - (D + 1), H), jnp.float32)], axis=0)

    z128 = jnp.zeros((NACC, H), jnp.float32)
    z16 = jnp.zeros((NACC, 16), jnp.float32)
    ones16 = jnp.ones((CHUNK, 16), jnp.float32)

    deg2 = _sc_deg(dstr, z16, ones16)
    xs1, dinv = _tc1(x, pl2, batch2, deg2, gn1w, gn1b, gn1ms, w1p)
    p1 = _sc_msg(xs1, srcr, dstr, z128)
    xs2, h1 = _tc2(p1, xs1, dinv, _row(b1), batch2,
                   _row(gn2_w), _row(gn2_b), _row(gn2_ms), W2)
    p2 = _sc_msg(xs2, srcr, dstr, z128)
    y = _tc3(p2, xs2, dinv, _row(b2), h1, batch2, _row(bn1_g), _row(bn1_b),
             fcW1, _row(fcb1), _row(bn2_g), _row(bn2_b), fcW2, _row(fcb2))
    return y


# speedup vs baseline: 16.3852x; 16.3852x over previous
import functools

import jax
import jax.numpy as jnp
from jax import lax
from jax.experimental import pallas as pl
from jax.experimental.pallas import tpu as pltpu
from jax.experimental.pallas import tpu_sc as plsc

N = 10000
E = 320000
D = 128
G = 64
H = 128
EPS = 1e-5

NC = 2
NS = 16
NW = NC * NS
CHUNK = 128
EPAD = 327680
EROWS = EPAD // CHUNK
TPW = EROWS // NW
TPC = EROWS // NS
SCW = 16
SCN = TPC // SCW
NACC = 10112
ZPW = NACC // NS
NSRC = 10048
FC = 128

_HI = lax.Precision.HIGHEST


def _mm(a, b):
    return lax.dot_general(a, b, (((1,), (0,)), ((), ())),
                           precision=_HI, preferred_element_type=jnp.float32)


def _mmT(a, b):
    return lax.dot_general(a, b, (((0,), (0,)), ((), ())),
                           precision=_HI, preferred_element_type=jnp.float32)


def _leaky(v):
    return jnp.where(v >= 0, v, 0.01 * v)


def _msg_body(xs_hbm, srcr_hbm, dstr_hbm, zeros_hbm, out_hbm,
              sidx, didx, row_a, row_b, acc, sem_a, sem_b, sem_i0, sem_i1):
    s = lax.axis_index("s")
    base = s * TPC
    pltpu.sync_copy(zeros_hbm.at[pl.ds(s * ZPW, ZPW)], acc.at[pl.ds(s * ZPW, ZPW)])

    sems_i = (sem_i0, sem_i1)

    def idx_start(g, b):
        pltpu.async_copy(srcr_hbm.at[pl.ds(base + g * SCW, SCW)], sidx.at[b],
                         sems_i[b])
        pltpu.async_copy(dstr_hbm.at[pl.ds(base + g * SCW, SCW)], didx.at[b],
                         sems_i[b])

    def idx_wait(g, b):
        pltpu.make_async_copy(srcr_hbm.at[pl.ds(base + g * SCW, SCW)],
                              sidx.at[b], sems_i[b]).wait()
        pltpu.make_async_copy(dstr_hbm.at[pl.ds(base + g * SCW, SCW)],
                              didx.at[b], sems_i[b]).wait()

    idx_start(0, 0)
    idx_start(1, 1)
    plsc.subcore_barrier()

    rows = (row_a, row_b)
    sems = (sem_a, sem_b)

    def g_start(b, i, r):
        pltpu.async_copy(xs_hbm.at[sidx.at[b, i]], rows[r], sems[r])

    def g_wait(b, i, r):
        pltpu.make_async_copy(xs_hbm.at[sidx.at[b, i]], rows[r], sems[r]).wait()

    def process(b):
        g_start(b, 0, 0)
        for i in range(SCW):
            if i + 1 < SCW:
                g_start(b, i + 1, (i + 1) % 2)
            g_wait(b, i, i % 2)
            pltpu.sync_copy(rows[i % 2], acc.at[didx.at[b, i]], add=True)

    def body(t, carry):
        g0 = 2 * t
        idx_wait(g0, 0)
        process(0)

        @pl.when(g0 + 2 < SCN)
        def _():
            idx_start(g0 + 2, 0)

        idx_wait(g0 + 1, 1)
        process(1)

        @pl.when(g0 + 3 < SCN)
        def _():
            idx_start(g0 + 3, 1)

        return carry

    lax.fori_loop(0, SCN // 2, body, 0)

    plsc.subcore_barrier()
    pltpu.sync_copy(acc.at[pl.ds(s * ZPW, ZPW)],
                    out_hbm.at[pl.ds(s * ZPW, ZPW)])


def _deg_body(dstr_hbm, zeros_hbm, ones_hbm, out_hbm, dst_v, ones_v, acc):
    c = lax.axis_index("c")
    s = lax.axis_index("s")
    wid = s * NC + c
    pltpu.sync_copy(zeros_hbm.at[pl.ds(s * ZPW, ZPW)], acc.at[pl.ds(s * ZPW, ZPW)])
    pltpu.sync_copy(dstr_hbm.at[pl.ds(wid * TPW, TPW)], dst_v)
    pltpu.sync_copy(ones_hbm, ones_v)
    plsc.subcore_barrier()

    def body(j, carry):
        pltpu.sync_copy(ones_v, acc.at[dst_v.at[j]], add=True)
        return carry

    lax.fori_loop(0, TPW, body, 0)

    plsc.subcore_barrier()
    pltpu.sync_copy(acc.at[pl.ds(s * ZPW, ZPW)],
                    out_hbm.at[c, pl.ds(s * ZPW, ZPW)])


@functools.lru_cache(maxsize=None)
def _sc_mesh(num_cores):
    return plsc.VectorSubcoreMesh(core_axis_name="c", subcore_axis_name="s",
                                  num_cores=num_cores, num_subcores=NS)


@functools.lru_cache(maxsize=None)
def _sc_msg_kernel():
    return pl.kernel(
        _msg_body,
        out_type=jax.ShapeDtypeStruct((NACC, H), jnp.float32),
        mesh=_sc_mesh(1),
        scratch_types=[
            pltpu.VMEM((2, SCW, CHUNK), jnp.int32),
            pltpu.VMEM((2, SCW, CHUNK), jnp.int32),
            pltpu.VMEM((CHUNK, H), jnp.float32),
            pltpu.VMEM((CHUNK, H), jnp.float32),
            pltpu.VMEM_SHARED((NACC, H), jnp.float32),
            pltpu.SemaphoreType.DMA,
            pltpu.SemaphoreType.DMA,
            pltpu.SemaphoreType.DMA,
            pltpu.SemaphoreType.DMA,
        ],
    )


def _sc_msg(xs, srcr, dstr, z128):
    return _sc_msg_kernel()(xs, srcr, dstr, z128)


def _sc_deg(dstr, z16, ones16):
    k = pl.kernel(
        _deg_body,
        out_type=jax.ShapeDtypeStruct((NC, NACC, 16), jnp.float32),
        mesh=_sc_mesh(NC),
        scratch_types=[
            pltpu.VMEM((TPW, CHUNK), jnp.int32),
            pltpu.VMEM((CHUNK, 16), jnp.float32),
            pltpu.VMEM_SHARED((NACC, 16), jnp.float32),
        ],
    )
    return k(dstr, z16, ones16)


ROWB = 2000
RB = N // ROWB


def _coef(m, m2, w, b, ms):
    var = m2 - (2.0 * ms - ms * ms) * m * m
    rstd = lax.rsqrt(var + EPS)
    a = w * rstd
    return a, b - a * ms * m


def _acc2(i, va, vb, ra, rb):
    @pl.when(i == 0)
    def _():
        ra[...] = va
        rb[...] = vb

    @pl.when(i != 0)
    def _():
        ra[...] = ra[...] + va
        rb[...] = rb[...] + vb


def _stats1_body(x_ref, pl_ref, sts_ref, mx_ref, m2x_ref, mp_ref, m2p_ref):
    i = pl.program_id(0)
    x = x_ref[...]
    p = pl_ref[...]
    sts = sts_ref[...]
    _acc2(i, _mmT(sts, x), _mmT(sts, x * x), mx_ref, m2x_ref)
    _acc2(i, _mmT(sts, p), _mmT(sts, p * p), mp_ref, m2p_ref)


def _apply1_body(x_ref, pl_ref, st_ref, deg_ref, mx_ref, m2x_ref, mp_ref,
                 m2p_ref, gnwx_ref, gnbx_ref, gnmsx_ref, gnwp_ref, gnbp_ref,
                 gnmsp_ref, w1x_ref, w1p_ref, xs_ref, dinv_ref):
    st = st_ref[...]
    ax, bx = _coef(mx_ref[...], m2x_ref[...], gnwx_ref[...], gnbx_ref[...],
                   gnmsx_ref[...])
    ap, bp = _coef(mp_ref[...], m2p_ref[...], gnwp_ref[...], gnbp_ref[...],
                   gnmsp_ref[...])
    gnx = _mm(st, ax) * x_ref[...] + _mm(st, bx)
    gnp = _mm(st, ap) * pl_ref[...] + _mm(st, bp)
    xw = _mm(gnx, w1x_ref[...]) + gnp * w1p_ref[...]
    dinv = lax.rsqrt(deg_ref[...])
    xs_ref[...] = xw * dinv
    dinv_ref[...] = dinv


def _hstats_body(p_ref, xs_ref, dinv_ref, b_ref, sts_ref,
                 h_ref, mh_ref, m2h_ref):
    i = pl.program_id(0)
    h = _leaky(dinv_ref[...] * (p_ref[...] + xs_ref[...]) + b_ref[...])
    h_ref[...] = h
    sts = sts_ref[...]
    _acc2(i, _mmT(sts, h), _mmT(sts, h * h), mh_ref, m2h_ref)


def _apply2_body(h_ref, st_ref, dinv_ref, mh_ref, m2h_ref, gnw_ref, gnb_ref,
                 gnms_ref, w2_ref, xs2_ref):
    st = st_ref[...]
    a, b = _coef(mh_ref[...], m2h_ref[...], gnw_ref[...], gnb_ref[...],
                 gnms_ref[...])
    gn = _mm(st, a) * h_ref[...] + _mm(st, b)
    xs2_ref[...] = _mm(gn, w2_ref[...]) * dinv_ref[...]


def _bn(v, g, b):
    m = jnp.mean(v, axis=0, keepdims=True)
    var = jnp.mean((v - m) ** 2, axis=0, keepdims=True)
    return g * (v - m) * lax.rsqrt(var + EPS) + b


def _final_body(p_ref, xs2_ref, dinv_ref, b2_ref, h1_ref, sts_ref,
                bn1g_ref, bn1b_ref, fw1_ref, fb1_ref, bn2g_ref, bn2b_ref,
                fw2_ref, fb2_ref, y_ref, pool1_ref, pool2_ref):
    i = pl.program_id(0)
    h2 = _leaky(dinv_ref[...] * (p_ref[...] + xs2_ref[...]) + b2_ref[...])
    sts = sts_ref[...]
    _acc2(i, _mmT(sts, h1_ref[...]), _mmT(sts, h2), pool1_ref, pool2_ref)

    @pl.when(i == RB - 1)
    def _():
        pooled = jnp.concatenate([pool1_ref[...], pool2_ref[...]], axis=1)
        y1 = _leaky(_mm(_bn(pooled, bn1g_ref[...], bn1b_ref[...]),
                        fw1_ref[...]) + fb1_ref[...])
        y_ref[...] = (_mm(_bn(y1, bn2g_ref[...], bn2b_ref[...]),
                          fw2_ref[...]) + fb2_ref[...])


def _rblk(cols):
    return pl.BlockSpec((ROWB, cols), lambda i: (i, 0))


def _full(shape):
    return pl.BlockSpec(shape, lambda i: (0, 0))


def _f32(shape):
    return jax.ShapeDtypeStruct(shape, jnp.float32)


def _tc_stats1(x, pl2, sts):
    return pl.pallas_call(
        _stats1_body,
        grid=(RB,),
        in_specs=[_rblk(D), _rblk(1), _rblk(G)],
        out_specs=[_full((G, D)), _full((G, D)), _full((G, 1)), _full((G, 1))],
        out_shape=[_f32((G, D)), _f32((G, D)), _f32((G, 1)), _f32((G, 1))],
    )(x, pl2, sts)


def _tc_apply1(x, pl2, st, deg, stats, gparams, w1x, w1row):
    return pl.pallas_call(
        _apply1_body,
        grid=(RB,),
        in_specs=[_rblk(D), _rblk(1), _rblk(G), _rblk(1),
                  _full((G, D)), _full((G, D)), _full((G, 1)), _full((G, 1)),
                  _full((1, D)), _full((1, D)), _full((1, D)),
                  _full((1, 1)), _full((1, 1)), _full((1, 1)),
                  _full((D, H)), _full((1, H))],
        out_specs=[_rblk(H), _rblk(1)],
        out_shape=[_f32((N, H)), _f32((N, 1))],
    )(x, pl2, st, deg, *stats, *gparams, w1x, w1row)


def _tc_hstats(p, xs, dinv, br, sts):
    return pl.pallas_call(
        _hstats_body,
        grid=(RB,),
        in_specs=[_rblk(H), _rblk(H), _rblk(1), _full((1, H)), _rblk(G)],
        out_specs=[_rblk(H), _full((G, H)), _full((G, H))],
        out_shape=[_f32((N, H)), _f32((G, H)), _f32((G, H))],
    )(p, xs, dinv, br, sts)


def _tc_apply2(h1, st, dinv, mh, m2h, gnw, gnb, gnms, w2):
    return pl.pallas_call(
        _apply2_body,
        grid=(RB,),
        in_specs=[_rblk(H), _rblk(G), _rblk(1),
                  _full((G, H)), _full((G, H)),
                  _full((1, H)), _full((1, H)), _full((1, H)),
                  _full((H, H))],
        out_specs=_rblk(H),
        out_shape=_f32((N, H)),
    )(h1, st, dinv, mh, m2h, gnw, gnb, gnms, w2)


def _tc_final(p2, xs2, dinv, b2r, h1, sts, bn1g, bn1b, fw1, fb1,
              bn2g, bn2b, fw2, fb2):
    return pl.pallas_call(
        _final_body,
        grid=(RB,),
        in_specs=[_rblk(H), _rblk(H), _rblk(1), _full((1, H)), _rblk(H),
                  _rblk(G), _full((1, 2 * H)), _full((1, 2 * H)),
                  _full((2 * H, FC)), _full((1, FC)), _full((1, FC)),
                  _full((1, FC)), _full((FC, 1)), _full((1, 1))],
        out_specs=_full((G, 1)),
        out_shape=_f32((G, 1)),
        scratch_shapes=[pltpu.VMEM((G, H), jnp.float32),
                        pltpu.VMEM((G, H), jnp.float32)],
    )(p2, xs2, dinv, b2r, h1, sts, bn1g, bn1b, fw1, fb1, bn2g, bn2b,
      fw2, fb2)


def _row(v):
    return v.reshape(1, -1).astype(jnp.float32)


def kernel(x, pLDDT, edge_index, batch, gn1_w, gn1_b, gn1_ms, W1, b1,
           gn2_w, gn2_b, gn2_ms, W2, b2, bn1_g, bn1_b, fcW1, fcb1,
           bn2_g, bn2_b, fcW2, fcb2):
    npad = EPAD - E
    pad_src = N + (jnp.arange(npad, dtype=jnp.int32) % (NSRC - N))
    pad_dst = N + (jnp.arange(npad, dtype=jnp.int32) % (NACC - N))
    srcr = jnp.concatenate([edge_index[0], pad_src]).reshape(EROWS, CHUNK)
    dstr = jnp.concatenate([edge_index[1], pad_dst]).reshape(EROWS, CHUNK)

    st = (batch.reshape(N, 1) == jnp.arange(G, dtype=batch.dtype)
          .reshape(1, G)).astype(jnp.float32)
    sts = st / jnp.maximum(jnp.sum(st, axis=0, keepdims=True), 1.0)
    pl2 = pLDDT.reshape(N, 1)

    z128 = jnp.zeros((NACC, H), jnp.float32)
    z16 = jnp.zeros((NACC, 16), jnp.float32)
    ones16 = jnp.ones((CHUNK, 16), jnp.float32)

    zrows = jnp.zeros((NSRC - N, H), jnp.float32)

    deg2 = _sc_deg(dstr, z16, ones16)
    deg = (deg2[0, :N, 0:1] + deg2[1, :N, 0:1]) + 1.0
    stats1 = _tc_stats1(x, pl2, sts)
    gparams1 = (_row(gn1_w[:D]), _row(gn1_b[:D]), _row(gn1_ms[:D]),
                gn1_w[D:].reshape(1, 1), gn1_b[D:].reshape(1, 1),
                gn1_ms[D:].reshape(1, 1))
    xs1, dinv = _tc_apply1(x, pl2, st, deg, stats1, gparams1,
                           W1[:D], W1[D:].reshape(1, H))
    p1 = _sc_msg(jnp.concatenate([xs1, zrows], axis=0), srcr, dstr, z128)
    h1, mh, m2h = _tc_hstats(p1, xs1, dinv, _row(b1), sts)
    xs2 = _tc_apply2(h1, st, dinv, mh, m2h, _row(gn2_w), _row(gn2_b),
                     _row(gn2_ms), W2)
    p2 = _sc_msg(jnp.concatenate([xs2, zrows], axis=0), srcr, dstr, z128)
    y = _tc_final(p2, xs2, dinv, _row(b2), h1, sts, _row(bn1_g), _row(bn1_b),
                  fcW1, _row(fcb1), _row(bn2_g), _row(bn2_b), fcW2, _row(fcb2))
    return y
